# baseline (speedup 1.0000x reference)
import functools

import jax
import jax.numpy as jnp
from jax import lax
from jax.experimental import pallas as pl
from jax.experimental.pallas import tpu as pltpu
from jax.experimental.pallas import tpu_sc as plsc

N = 10000
NP = 10240
D = 128
G = 64
E = 320000
NTILES = 32
CHUNK = 128
NCHUNKS = 80
EPT = NCHUNKS * CHUNK
ROWS_PER_TILE = NP // 16
NBLK = 10
BLK = NP // NBLK

_mesh = plsc.VectorSubcoreMesh(core_axis_name="c", subcore_axis_name="s")


def _zero_rows(buf, nrows):
    zeros16 = jnp.zeros((16,), jnp.float32)

    def body(r, _):
        for j in range(buf.shape[1] // 16):
            buf[r, pl.ds(j * 16, 16)] = zeros16
        return 0

    lax.fori_loop(0, nrows, body, 0)


DW = 4


@functools.partial(
    pl.kernel,
    out_type=jax.ShapeDtypeStruct((2, NP, DW), jnp.float32),
    mesh=_mesh,
    scratch_types=[
        pltpu.VMEM((NCHUNKS, CHUNK), jnp.int32),
        pltpu.VMEM((CHUNK, DW), jnp.float32),
        pltpu.VMEM_SHARED((NP, DW), jnp.float32),
    ],
)
def _deg_kernel(dst_hbm, ones_hbm, zer_hbm, degp_hbm, idx_v, ones_v, deg_sh):
    cid = lax.axis_index("c")
    sid = lax.axis_index("s")
    wid = cid * 16 + sid

    pltpu.sync_copy(zer_hbm, deg_sh.at[pl.ds(sid * ROWS_PER_TILE, ROWS_PER_TILE)])
    pltpu.sync_copy(ones_hbm, ones_v)
    plsc.subcore_barrier()

    pltpu.sync_copy(dst_hbm.at[wid], idx_v)

    def step(j, _):
        pltpu.sync_copy(ones_v, deg_sh.at[idx_v.at[j]], add=True)
        return 0

    lax.fori_loop(0, NCHUNKS, step, 0)
    plsc.subcore_barrier()

    pltpu.sync_copy(
        deg_sh.at[pl.ds(sid * ROWS_PER_TILE, ROWS_PER_TILE)],
        degp_hbm.at[cid, pl.ds(sid * ROWS_PER_TILE, ROWS_PER_TILE)],
    )


@functools.partial(
    pl.kernel,
    out_type=jax.ShapeDtypeStruct((2, NP, D), jnp.float32),
    mesh=_mesh,
    scratch_types=[
        pltpu.VMEM((NCHUNKS // 2, CHUNK), jnp.int32),
        pltpu.VMEM((NCHUNKS // 2, CHUNK), jnp.int32),
    ] + [pltpu.VMEM((CHUNK, D), jnp.float32)] * 2
      + [pltpu.VMEM_SHARED((NP, D), jnp.float32)]
      + [pltpu.SemaphoreType.DMA] * 2,
)
def _segsum_kernel(y_hbm, src_hbm, dst_hbm, zp_hbm, src_v, dst_v,
                   b0, b1, z_sh, g0, g1):
    cid = lax.axis_index("c")
    sid = lax.axis_index("s")
    wid = cid * 16 + sid
    bufs = (b0, b1)
    gsem = (g0, g1)

    _zero_rows(b0, CHUNK)

    def zslice(t, _):
        pltpu.sync_copy(
            b0,
            z_sh.at[pl.ds(sid * ROWS_PER_TILE + t * CHUNK, CHUNK)],
        )
        return 0

    lax.fori_loop(0, ROWS_PER_TILE // CHUNK, zslice, 0)
    plsc.subcore_barrier()

    for ph in range(2):
        pltpu.sync_copy(src_hbm.at[wid, pl.ds(ph * (NCHUNKS // 2), NCHUNKS // 2)],
                        src_v)
        pltpu.sync_copy(dst_hbm.at[wid, pl.ds(ph * (NCHUNKS // 2), NCHUNKS // 2)],
                        dst_v)

        def group(t, _):
            j0 = t * 2
            d0 = pltpu.async_copy(y_hbm.at[src_v.at[j0]], bufs[0], gsem[0])
            d1 = pltpu.async_copy(y_hbm.at[src_v.at[j0 + 1]], bufs[1], gsem[1])
            d0.wait()
            pltpu.sync_copy(bufs[0], z_sh.at[dst_v.at[j0]], add=True)
            d1.wait()
            pltpu.sync_copy(bufs[1], z_sh.at[dst_v.at[j0 + 1]], add=True)
            return 0

        lax.fori_loop(0, NCHUNKS // 4, group, 0)
    plsc.subcore_barrier()

    pltpu.sync_copy(
        z_sh.at[pl.ds(sid * ROWS_PER_TILE, ROWS_PER_TILE)],
        zp_hbm.at[cid, pl.ds(sid * ROWS_PER_TILE, ROWS_PER_TILE)],
    )


def _tc_first_body(fts_b, w_b, degp_b, y_b, dinv_b):
    i = pl.program_id(0)
    deg = degp_b[0, :, :1] + degp_b[1, :, :1] + 1.0
    rows = i * BLK + lax.broadcasted_iota(jnp.int32, (BLK, 1), 0)
    dinv = jnp.where(rows < N, lax.rsqrt(deg), 0.0)
    dinv_b[...] = dinv
    y_b[...] = jnp.dot(fts_b[...], w_b[...], preferred_element_type=jnp.float32) * dinv


def _tc_first(fts_p, W1, degp):
    return pl.pallas_call(
        _tc_first_body,
        grid=(NBLK,),
        in_specs=[
            pl.BlockSpec((BLK, D), lambda i: (i, 0)),
            pl.BlockSpec((D, D), lambda i: (0, 0)),
            pl.BlockSpec((2, BLK, DW), lambda i: (0, i, 0)),
        ],
        out_specs=[
            pl.BlockSpec((BLK, D), lambda i: (i, 0)),
            pl.BlockSpec((BLK, 1), lambda i: (i, 0)),
        ],
        out_shape=[
            jax.ShapeDtypeStruct((NP, D), jnp.float32),
            jax.ShapeDtypeStruct((NP, 1), jnp.float32),
        ],
    )(fts_p, W1, degp)


def _tc_mid_body(z_b, y_b, dinv_b, b_b, w_b, out_b):
    h = (z_b[0] + z_b[1] + y_b[...]) * dinv_b[...] + b_b[...]
    h = jnp.maximum(h, 0.0)
    out_b[...] = jnp.dot(h, w_b[...], preferred_element_type=jnp.float32) * dinv_b[...]


def _tc_mid(z, y, dinv, b, Wn):
    return pl.pallas_call(
        _tc_mid_body,
        grid=(NBLK,),
        in_specs=[
            pl.BlockSpec((2, BLK, D), lambda i: (0, i, 0)),
            pl.BlockSpec((BLK, D), lambda i: (i, 0)),
            pl.BlockSpec((BLK, 1), lambda i: (i, 0)),
            pl.BlockSpec((1, D), lambda i: (0, 0)),
            pl.BlockSpec((D, D), lambda i: (0, 0)),
        ],
        out_specs=pl.BlockSpec((BLK, D), lambda i: (i, 0)),
        out_shape=jax.ShapeDtypeStruct((NP, D), jnp.float32),
    )(z, y, dinv, b.reshape(1, D), Wn)


def _tc_final_body(z_b, y_b, dinv_b, b_b, batch_b, fw_b, fb_b, out_b, sums, cnt):
    i = pl.program_id(0)

    @pl.when(i == 0)
    def _():
        sums[...] = jnp.zeros_like(sums)
        cnt[...] = jnp.zeros_like(cnt)

    h = (z_b[0] + z_b[1] + y_b[...]) * dinv_b[...] + b_b[...]
    onehot = (batch_b[...] == lax.broadcasted_iota(jnp.int32, (BLK, G), 1)
              ).astype(jnp.float32)
    sums[...] += lax.dot_general(
        onehot, h, (((0,), (0,)), ((), ())), preferred_element_type=jnp.float32)
    cnt[...] += jnp.broadcast_to(jnp.sum(onehot, axis=0)[:, None], (G, D))

    @pl.when(i == NBLK - 1)
    def _():
        pooled = sums[...] / jnp.maximum(cnt[...], 1.0)
        out_b[...] = jnp.dot(pooled, fw_b[...],
                             preferred_element_type=jnp.float32) + fb_b[...]


def _tc_final(z, y, dinv, b5, batch_p, fW, fb):
    return pl.pallas_call(
        _tc_final_body,
        grid=(NBLK,),
        in_specs=[
            pl.BlockSpec((2, BLK, D), lambda i: (0, i, 0)),
            pl.BlockSpec((BLK, D), lambda i: (i, 0)),
            pl.BlockSpec((BLK, 1), lambda i: (i, 0)),
            pl.BlockSpec((1, D), lambda i: (0, 0)),
            pl.BlockSpec((BLK, 1), lambda i: (i, 0)),
            pl.BlockSpec((D, D), lambda i: (0, 0)),
            pl.BlockSpec((1, D), lambda i: (0, 0)),
        ],
        out_specs=pl.BlockSpec((G, D), lambda i: (0, 0)),
        out_shape=jax.ShapeDtypeStruct((G, D), jnp.float32),
        scratch_shapes=[
            pltpu.VMEM((G, D), jnp.float32),
            pltpu.VMEM((G, D), jnp.float32),
        ],
    )(z, y, dinv, b5.reshape(1, D), batch_p, fW, fb.reshape(1, D))


def kernel(fts, adj, batch, W1, b1, W2, b2, W3, b3, W4, b4, W5, b5, fW, fb):
    pad_e = NTILES * EPT - E
    src_t = jnp.concatenate(
        [adj[0], jnp.full((pad_e,), N, jnp.int32)]).reshape(NTILES, NCHUNKS, CHUNK)
    dst_t = jnp.concatenate(
        [adj[1], jnp.full((pad_e,), N, jnp.int32)]).reshape(NTILES, NCHUNKS, CHUNK)
    fts_p = jnp.concatenate([fts, jnp.zeros((NP - N, D), jnp.float32)])
    batch_p = jnp.concatenate(
        [batch, jnp.full((NP - N,), G, jnp.int32)]).reshape(NP, 1)

    degp = _deg_kernel(dst_t, jnp.ones((CHUNK, DW), jnp.float32),
                       jnp.zeros((ROWS_PER_TILE, DW), jnp.float32))
    y, dinv = _tc_first(fts_p, W1, degp)
    Ws = [W2, W3, W4, W5]
    bs = [b1, b2, b3, b4]
    for i in range(4):
        z = _segsum_kernel(y, src_t, dst_t)
        y = _tc_mid(z, y, dinv, bs[i], Ws[i])
    z = _segsum_kernel(y, src_t, dst_t)
    return _tc_final(z, y, dinv, b5, batch_p, fW, fb)


# speedup vs baseline: 1.0786x; 1.0786x over previous
import functools

import jax
import jax.numpy as jnp
from jax import lax
from jax.experimental import pallas as pl
from jax.experimental.pallas import tpu as pltpu
from jax.experimental.pallas import tpu_sc as plsc

N = 10000
NP = 10240
D = 128
G = 64
E = 320000
NTILES = 32
CHUNK = 128
NCHUNKS = 80
GSZ = 256
NGRP = 40
EPT = NCHUNKS * CHUNK
ROWS_PER_TILE = NP // 16
NBLK = 10
BLK = NP // NBLK

_mesh = plsc.VectorSubcoreMesh(core_axis_name="c", subcore_axis_name="s")


def _zero_rows(buf, nrows):
    zeros16 = jnp.zeros((16,), jnp.float32)

    def body(r, _):
        for j in range(buf.shape[1] // 16):
            buf[r, pl.ds(j * 16, 16)] = zeros16
        return 0

    lax.fori_loop(0, nrows, body, 0)


DW = 4


@functools.partial(
    pl.kernel,
    out_type=jax.ShapeDtypeStruct((2, NP, DW), jnp.float32),
    mesh=_mesh,
    scratch_types=[
        pltpu.VMEM((NCHUNKS, CHUNK), jnp.int32),
        pltpu.VMEM((CHUNK, DW), jnp.float32),
        pltpu.VMEM_SHARED((NP, DW), jnp.float32),
    ],
)
def _deg_kernel(dst_hbm, ones_hbm, zer_hbm, degp_hbm, idx_v, ones_v, deg_sh):
    cid = lax.axis_index("c")
    sid = lax.axis_index("s")
    wid = cid * 16 + sid

    pltpu.sync_copy(zer_hbm, deg_sh.at[pl.ds(sid * ROWS_PER_TILE, ROWS_PER_TILE)])
    pltpu.sync_copy(ones_hbm, ones_v)
    plsc.subcore_barrier()

    pltpu.sync_copy(dst_hbm.at[wid], idx_v)

    def step(j, _):
        pltpu.sync_copy(ones_v, deg_sh.at[idx_v.at[j]], add=True)
        return 0

    lax.fori_loop(0, NCHUNKS, step, 0)
    plsc.subcore_barrier()

    pltpu.sync_copy(
        deg_sh.at[pl.ds(sid * ROWS_PER_TILE, ROWS_PER_TILE)],
        degp_hbm.at[cid, pl.ds(sid * ROWS_PER_TILE, ROWS_PER_TILE)],
    )


@functools.partial(
    pl.kernel,
    out_type=jax.ShapeDtypeStruct((2, NP, D), jnp.float32),
    mesh=_mesh,
    scratch_types=[
        pltpu.VMEM((NGRP // 2 * GSZ,), jnp.int32),
        pltpu.VMEM((NGRP // 2 * GSZ,), jnp.int32),
    ] + [pltpu.VMEM((GSZ, D), jnp.float32)]
      + [pltpu.VMEM_SHARED((NP, D), jnp.float32)]
      + [pltpu.SemaphoreType.DMA],
)
def _segsum_kernel(y_hbm, src_hbm, dst_hbm, zp_hbm, src_v, dst_v,
                   buf, z_sh, gsem):
    cid = lax.axis_index("c")
    sid = lax.axis_index("s")
    wid = cid * 16 + sid

    _zero_rows(buf, GSZ)
    pltpu.sync_copy(buf, z_sh.at[pl.ds(sid * ROWS_PER_TILE, GSZ)])
    pltpu.sync_copy(buf.at[pl.ds(0, ROWS_PER_TILE - GSZ)],
                    z_sh.at[pl.ds(sid * ROWS_PER_TILE + GSZ,
                                  ROWS_PER_TILE - GSZ)])
    plsc.subcore_barrier()

    for ph in range(2):
        pltpu.sync_copy(src_hbm.at[wid, ph], src_v)
        pltpu.sync_copy(dst_hbm.at[wid, ph], dst_v)

        def group(t, _):
            sl = pl.ds(t * GSZ, GSZ)
            pltpu.async_copy(y_hbm.at[src_v.at[sl]], buf, gsem).wait()
            pltpu.sync_copy(buf, z_sh.at[dst_v.at[sl]], add=True)
            return 0

        lax.fori_loop(0, NGRP // 2, group, 0)
    plsc.subcore_barrier()

    pltpu.sync_copy(
        z_sh.at[pl.ds(sid * ROWS_PER_TILE, ROWS_PER_TILE)],
        zp_hbm.at[cid, pl.ds(sid * ROWS_PER_TILE, ROWS_PER_TILE)],
    )


def _tc_first_body(fts_b, w_b, degp_b, y_b, dinv_b):
    i = pl.program_id(0)
    deg = degp_b[0, :, :1] + degp_b[1, :, :1] + 1.0
    rows = i * BLK + lax.broadcasted_iota(jnp.int32, (BLK, 1), 0)
    dinv = jnp.where(rows < N, lax.rsqrt(deg), 0.0)
    dinv_b[...] = dinv
    y_b[...] = jnp.dot(fts_b[...], w_b[...], preferred_element_type=jnp.float32) * dinv


def _tc_first(fts_p, W1, degp):
    return pl.pallas_call(
        _tc_first_body,
        grid=(NBLK,),
        in_specs=[
            pl.BlockSpec((BLK, D), lambda i: (i, 0)),
            pl.BlockSpec((D, D), lambda i: (0, 0)),
            pl.BlockSpec((2, BLK, DW), lambda i: (0, i, 0)),
        ],
        out_specs=[
            pl.BlockSpec((BLK, D), lambda i: (i, 0)),
            pl.BlockSpec((BLK, 1), lambda i: (i, 0)),
        ],
        out_shape=[
            jax.ShapeDtypeStruct((NP, D), jnp.float32),
            jax.ShapeDtypeStruct((NP, 1), jnp.float32),
        ],
    )(fts_p, W1, degp)


def _tc_mid_body(z_b, y_b, dinv_b, b_b, w_b, out_b):
    h = (z_b[0] + z_b[1] + y_b[...]) * dinv_b[...] + b_b[...]
    h = jnp.maximum(h, 0.0)
    out_b[...] = jnp.dot(h, w_b[...], preferred_element_type=jnp.float32) * dinv_b[...]


def _tc_mid(z, y, dinv, b, Wn):
    return pl.pallas_call(
        _tc_mid_body,
        grid=(NBLK,),
        in_specs=[
            pl.BlockSpec((2, BLK, D), lambda i: (0, i, 0)),
            pl.BlockSpec((BLK, D), lambda i: (i, 0)),
            pl.BlockSpec((BLK, 1), lambda i: (i, 0)),
            pl.BlockSpec((1, D), lambda i: (0, 0)),
            pl.BlockSpec((D, D), lambda i: (0, 0)),
        ],
        out_specs=pl.BlockSpec((BLK, D), lambda i: (i, 0)),
        out_shape=jax.ShapeDtypeStruct((NP, D), jnp.float32),
    )(z, y, dinv, b.reshape(1, D), Wn)


def _tc_final_body(z_b, y_b, dinv_b, b_b, batch_b, fw_b, fb_b, out_b, sums, cnt):
    i = pl.program_id(0)

    @pl.when(i == 0)
    def _():
        sums[...] = jnp.zeros_like(sums)
        cnt[...] = jnp.zeros_like(cnt)

    h = (z_b[0] + z_b[1] + y_b[...]) * dinv_b[...] + b_b[...]
    onehot = (batch_b[...] == lax.broadcasted_iota(jnp.int32, (BLK, G), 1)
              ).astype(jnp.float32)
    sums[...] += lax.dot_general(
        onehot, h, (((0,), (0,)), ((), ())), preferred_element_type=jnp.float32)
    cnt[...] += jnp.broadcast_to(jnp.sum(onehot, axis=0)[:, None], (G, D))

    @pl.when(i == NBLK - 1)
    def _():
        pooled = sums[...] / jnp.maximum(cnt[...], 1.0)
        out_b[...] = jnp.dot(pooled, fw_b[...],
                             preferred_element_type=jnp.float32) + fb_b[...]


def _tc_final(z, y, dinv, b5, batch_p, fW, fb):
    return pl.pallas_call(
        _tc_final_body,
        grid=(NBLK,),
        in_specs=[
            pl.BlockSpec((2, BLK, D), lambda i: (0, i, 0)),
            pl.BlockSpec((BLK, D), lambda i: (i, 0)),
            pl.BlockSpec((BLK, 1), lambda i: (i, 0)),
            pl.BlockSpec((1, D), lambda i: (0, 0)),
            pl.BlockSpec((BLK, 1), lambda i: (i, 0)),
            pl.BlockSpec((D, D), lambda i: (0, 0)),
            pl.BlockSpec((1, D), lambda i: (0, 0)),
        ],
        out_specs=pl.BlockSpec((G, D), lambda i: (0, 0)),
        out_shape=jax.ShapeDtypeStruct((G, D), jnp.float32),
        scratch_shapes=[
            pltpu.VMEM((G, D), jnp.float32),
            pltpu.VMEM((G, D), jnp.float32),
        ],
    )(z, y, dinv, b5.reshape(1, D), batch_p, fW, fb.reshape(1, D))


def kernel(fts, adj, batch, W1, b1, W2, b2, W3, b3, W4, b4, W5, b5, fW, fb):
    pad_e = NTILES * EPT - E
    srcp = jnp.concatenate([adj[0], jnp.full((pad_e,), N, jnp.int32)])
    dstp = jnp.concatenate([adj[1], jnp.full((pad_e,), N, jnp.int32)])
    src_t = srcp.reshape(NTILES, 2, NGRP // 2 * GSZ)
    dst_t = dstp.reshape(NTILES, 2, NGRP // 2 * GSZ)
    dst_deg = dstp.reshape(NTILES, NCHUNKS, CHUNK)
    fts_p = jnp.concatenate([fts, jnp.zeros((NP - N, D), jnp.float32)])
    batch_p = jnp.concatenate(
        [batch, jnp.full((NP - N,), G, jnp.int32)]).reshape(NP, 1)

    degp = _deg_kernel(dst_deg, jnp.ones((CHUNK, DW), jnp.float32),
                       jnp.zeros((ROWS_PER_TILE, DW), jnp.float32))
    y, dinv = _tc_first(fts_p, W1, degp)
    Ws = [W2, W3, W4, W5]
    bs = [b1, b2, b3, b4]
    for i in range(4):
        z = _segsum_kernel(y, src_t, dst_t)
        y = _tc_mid(z, y, dinv, bs[i], Ws[i])
    z = _segsum_kernel(y, src_t, dst_t)
    return _tc_final(z, y, dinv, b5, batch_p, fW, fb)
